# baseline (speedup 1.0000x reference)
import dataclasses
import functools

import jax
import jax.numpy as jnp
from jax import lax
from jax.experimental import pallas as pl
from jax.experimental.pallas import tpu as pltpu
from jax.experimental.pallas import tpu_sc as plsc

N = 10000
D_IN = 128
HID = 128
OUT = 64

N_PAD = 10240
DUMMY = N
NC, NS = 2, 16
CHUNKS = 80
CW = 128
E_PW = CHUNKS * CW
E_PAD = NC * NS * E_PW
ROWS_PS = N_PAD // NS
RB = 512
GRID = N_PAD // RB

CK_SET = E_PAD // CW
C_FAST, C_SLOW = 148, 12
assert NS * (C_FAST + C_SLOW) == CK_SET
assert C_FAST % 2 == 0 and C_SLOW % 2 == 0 and C_SLOW >= 2
FAST_CORE = 0


def _sc_compiler_params():
    cp = pltpu.CompilerParams()
    if "needs_layout_passes" in pltpu.CompilerParams.__dataclass_fields__:
        cp = dataclasses.replace(cp, needs_layout_passes=False)
    return cp


NROW = N_PAD // CW
RPS = NROW // NS


def _rsqrt16(x):
    i = plsc.bitcast(x, jnp.int32)
    y = plsc.bitcast(jnp.int32(0x5F3759DF) - lax.shift_right_logical(i, 1),
                     jnp.float32)
    for _ in range(3):
        y = y * (1.5 - 0.5 * x * y * y)
    return y


@functools.cache
def _get_deg_kernel():
    mesh = plsc.VectorSubcoreMesh(core_axis_name="c", subcore_axis_name="s")
    return functools.partial(
        pl.kernel,
        mesh=mesh,
        compiler_params=_sc_compiler_params(),
        out_type=[jax.ShapeDtypeStruct((3, N_PAD, HID), jnp.float32),
                  jax.ShapeDtypeStruct((3, N_PAD, HID), jnp.float32)],
        scratch_types=[
            pltpu.VMEM((CHUNKS, CW), jnp.int32),
            pltpu.VMEM((CHUNKS, CW), jnp.float32),
            pltpu.VMEM((NROW, CW), jnp.float32),
            pltpu.VMEM((NROW, CW), jnp.float32),
            pltpu.VMEM((NROW, CW), jnp.float32),
            pltpu.VMEM((CW, CW), jnp.float32),
            pltpu.VMEM((3, NROW), jnp.int32),
            pltpu.VMEM_SHARED((3 * NROW, CW), jnp.float32),
        ],
    )(_deg_body)


def _deg_body(dst_hbm, ew_hbm, dis_hbm, selfw_hbm,
              dstv, ewv, a0, a1, a2, bcast, idxv, acc_sh):
    c = lax.axis_index("c")
    s = lax.axis_index("s")
    accs = [a0, a1, a2]

    @pl.loop(0, NROW)
    def _(i):
        z = jnp.zeros((16,), jnp.float32)
        for f in range(8):
            a0[i, pl.ds(16 * f, 16)] = z
            a1[i, pl.ds(16 * f, 16)] = z
            a2[i, pl.ds(16 * f, 16)] = z

    iota = jnp.arange(16, dtype=jnp.int32)
    for k in range(3):
        for m in range(NROW // 16):
            idxv[k, pl.ds(16 * m, 16)] = iota + (k * NROW + 16 * m)

    pltpu.sync_copy(a0.at[pl.ds(0, 15)],
                    acc_sh.at[pl.ds(s * (3 * NROW // NS), 3 * NROW // NS)])
    plsc.subcore_barrier()

    for k in range(3):
        for p in range(2):
            pltpu.sync_copy(
                dst_hbm.at[k, pl.ds(s * 2 * CHUNKS + p * CHUNKS, CHUNKS)],
                dstv)
            pltpu.sync_copy(
                ew_hbm.at[k, pl.ds(s * 2 * CHUNKS + p * CHUNKS, CHUNKS)],
                ewv)

            @pl.loop(0, CHUNKS)
            def _(j):
                for t in range(8):
                    d16 = dstv[j, pl.ds(16 * t, 16)]
                    w16 = ewv[j, pl.ds(16 * t, 16)]
                    plsc.addupdate_scatter(
                        accs[k],
                        [lax.shift_right_logical(d16, 7),
                         jnp.bitwise_and(d16, 127)],
                        w16)

    for k in range(3):
        pltpu.sync_copy(accs[k], acc_sh.at[idxv.at[k]], add=True)
    plsc.subcore_barrier()

    def emit(out_hbm, square):
        for k in range(3):
            pltpu.sync_copy(acc_sh.at[pl.ds(k * NROW + s * RPS, RPS)],
                            ewv.at[pl.ds(0, RPS)])

            @pl.loop(0, RPS)
            def _(r):
                @pl.loop(0, 8)
                def _(f):
                    x = ewv[r, pl.ds(16 * f, 16)] + 1.0
                    y = _rsqrt16(x)
                    if square:
                        y = y * y
                    ewv[CHUNKS - 1, pl.ds(0, 16)] = y

                    @pl.loop(0, 16)
                    def _(l):
                        v = plsc.load_gather(
                            ewv, [jnp.full((16,), CHUNKS - 1, jnp.int32),
                                  jnp.full((16,), l, jnp.int32)])

                        @pl.loop(0, 8)
                        def _(g):
                            bcast[16 * f + l, pl.ds(16 * g, 16)] = v

                pltpu.sync_copy(
                    bcast, out_hbm.at[k, pl.ds((s * RPS + r) * CW, CW)])

    @pl.when(c == FAST_CORE)
    def _():
        emit(dis_hbm, False)

    @pl.when(c != FAST_CORE)
    def _():
        emit(selfw_hbm, True)


@functools.cache
def _get_prop_kernel():
    mesh = plsc.VectorSubcoreMesh(core_axis_name="c", subcore_axis_name="s")
    return functools.partial(
        pl.kernel,
        mesh=mesh,
        compiler_params=_sc_compiler_params(),
        out_type=jax.ShapeDtypeStruct((NC, 3, N_PAD, HID), jnp.float32),
        scratch_types=[
            pltpu.VMEM((2, CW), jnp.int32),
            pltpu.VMEM((2, CW), jnp.int32),
            pltpu.VMEM((2, CW), jnp.float32),
            pltpu.VMEM((CW, HID), jnp.float32),
            pltpu.VMEM((CW, HID), jnp.float32),
            pltpu.VMEM_SHARED((N_PAD, HID), jnp.float32),
            pltpu.SemaphoreType.DMA,
            pltpu.SemaphoreType.DMA,
            pltpu.SemaphoreType.DMA,
            pltpu.SemaphoreType.DMA,
            pltpu.SemaphoreType.DMA,
            pltpu.SemaphoreType.DMA,
        ],
    )(_prop_body)


def _scale_chunk(buf, ewv, b):
    bfull = jnp.full((16,), b, jnp.int32)

    @pl.loop(0, CW)
    def _(e):
        w = plsc.load_gather(ewv, [bfull, jnp.full((16,), e, jnp.int32)])
        for f in range(8):
            sl = (e, pl.ds(16 * f, 16))
            buf[sl] = buf[sl] * w


def _prop_body(hh0, hh1, hh2, hh0b, hh1b, hh2b, src_hbm, dst_hbm, ew_hbm,
               out_hbm, srcv, dstv, ewv, buf0, buf1, acc,
               ss0, ss1, sd0, sd1, sg0, sg1):
    c = lax.axis_index("c")
    s = lax.axis_index("s")
    base = s * ROWS_PS
    bufs = [buf0, buf1]
    sem_src = [ss0, ss1]
    sem_de = [sd0, sd1]
    sem_g = [sg0, sg1]

    for k in range(3):

        def src_copy(j, b, _k=k):
            pltpu.async_copy(src_hbm.at[_k, pl.ds(j, 1)],
                             srcv.at[pl.ds(b, 1)], sem_src[b])

        def src_wait(b):
            pltpu.make_async_copy(src_hbm.at[0, pl.ds(0, 1)],
                                  srcv.at[pl.ds(0, 1)], sem_src[b]).wait()

        def de_copy(j, b, _k=k):
            pltpu.async_copy(dst_hbm.at[_k, pl.ds(j, 1)],
                             dstv.at[pl.ds(b, 1)], sem_de[b])
            if _k > 0:
                pltpu.async_copy(ew_hbm.at[_k, pl.ds(j, 1)],
                                 ewv.at[pl.ds(b, 1)], sem_de[b])

        def de_wait(b, _k=k):
            for _ in range(2 if _k > 0 else 1):
                pltpu.make_async_copy(dst_hbm.at[0, pl.ds(0, 1)],
                                      dstv.at[pl.ds(0, 1)], sem_de[b]).wait()

        def process(b, _k=k):
            if _k > 0:
                _scale_chunk(bufs[b], ewv, b)
            pltpu.sync_copy(bufs[b], acc.at[dstv.at[b]], add=True)

        def pipeline(cbase, nch, hh):
            def gather_start(b):
                pltpu.async_copy(hh.at[srcv.at[b]], bufs[b], sem_g[b])

            def gather_wait(b):
                pltpu.make_async_copy(hh.at[srcv.at[b]], bufs[b],
                                      sem_g[b]).wait()
            src_copy(cbase, 0)
            de_copy(cbase, 0)
            src_copy(cbase + 1, 1)
            de_copy(cbase + 1, 1)
            src_wait(0)
            gather_start(0)

            @pl.loop(0, nch - 2, step=2)
            def _(j):
                gather_wait(0)
                src_wait(1)
                gather_start(1)
                src_copy(cbase + j + 2, 0)
                de_wait(0)
                process(0)
                de_copy(cbase + j + 2, 0)
                gather_wait(1)
                src_wait(0)
                gather_start(0)
                src_copy(cbase + j + 3, 1)
                de_wait(1)
                process(1)
                de_copy(cbase + j + 3, 1)

            gather_wait(0)
            src_wait(1)
            gather_start(1)
            de_wait(0)
            process(0)
            gather_wait(1)
            de_wait(1)
            process(1)

        with jax.named_scope(f"zero{k}"):
            @pl.loop(0, CW)
            def _(r):
                for f in range(8):
                    buf0[r, pl.ds(16 * f, 16)] = jnp.zeros((16,), jnp.float32)

            for m in range(ROWS_PS // CW):
                pltpu.sync_copy(buf0, acc.at[pl.ds(base + m * CW, CW)])
            plsc.subcore_barrier()

        with jax.named_scope(f"pipe{k}"):
            @pl.when(c == FAST_CORE)
            def _():
                pipeline(s * C_FAST, C_FAST, [hh0, hh1, hh2][k])

            @pl.when(c != FAST_CORE)
            def _():
                pipeline(NS * C_FAST + s * C_SLOW, C_SLOW,
                         [hh0b, hh1b, hh2b][k])

            plsc.subcore_barrier()

        with jax.named_scope(f"dump{k}"):
            pltpu.sync_copy(acc.at[pl.ds(base, ROWS_PS)],
                            out_hbm.at[c, k, pl.ds(base, ROWS_PS)])


def _m0_body(x_ref, w0_ref, dis_ref, h_ref, g0_ref, g1_ref, g2_ref,
             g0b_ref, g1b_ref, g2b_ref):
    h = lax.dot_general(x_ref[...], w0_ref[...], (((1,), (1,)), ((), ())),
                        preferred_element_type=jnp.float32)
    dis_b = dis_ref[...]
    h_ref[...] = h
    g0_ref[...] = dis_b[0] * h
    g1_ref[...] = dis_b[1] * h
    g2_ref[...] = dis_b[2] * h
    g0b_ref[...] = dis_b[0] * h
    g1b_ref[...] = dis_b[1] * h
    g2b_ref[...] = dis_b[2] * h


def _combine(part_ref, h_ref, dis_ref, selfw_ref, b_ref):
    p = part_ref[0] + part_ref[1]
    h = h_ref[...]
    xc = dis_ref[...] * p + selfw_ref[...] * h[None] + b_ref[...][None]
    xc = jnp.maximum(xc, 0.0)
    return jnp.concatenate([xc[0], xc[1], xc[2]], axis=-1)


def _mid_body(part_ref, h_ref, dis_ref, selfw_ref, b_ref, w_ref,
              hn_ref, g0_ref, g1_ref, g2_ref, g0b_ref, g1b_ref, g2b_ref):
    xcat = _combine(part_ref, h_ref, dis_ref, selfw_ref, b_ref)
    hn = lax.dot_general(xcat, w_ref[...], (((1,), (1,)), ((), ())),
                         preferred_element_type=jnp.float32)
    hn_ref[...] = hn
    dis_b = dis_ref[...]
    g0_ref[...] = dis_b[0] * hn
    g1_ref[...] = dis_b[1] * hn
    g2_ref[...] = dis_b[2] * hn
    g0b_ref[...] = dis_b[0] * hn
    g1b_ref[...] = dis_b[1] * hn
    g2b_ref[...] = dis_b[2] * hn


def _final_body(part_ref, h_ref, dis_ref, selfw_ref, b_ref, wc_ref, bc_ref,
                xcat_ref, ls_ref):
    xcat = _combine(part_ref, h_ref, dis_ref, selfw_ref, b_ref)
    xcat_ref[...] = xcat
    y = lax.dot_general(xcat, wc_ref[...], (((1,), (1,)), ((), ())),
                        preferred_element_type=jnp.float32)
    y = y + bc_ref[...]
    z = y - jnp.max(y, axis=1, keepdims=True)
    ls_ref[...] = z - jnp.log(jnp.sum(jnp.exp(z), axis=1, keepdims=True))


_f32 = jnp.float32


def _m0_call(xp, W0, dis_b):
    return pl.pallas_call(
        _m0_body,
        grid=(GRID,),
        in_specs=[
            pl.BlockSpec((RB, D_IN), lambda i: (i, 0)),
            pl.BlockSpec((HID, D_IN), lambda i: (0, 0)),
            pl.BlockSpec((3, RB, HID), lambda i: (0, i, 0)),
        ],
        out_specs=[pl.BlockSpec((RB, HID), lambda i: (i, 0))] * 7,
        out_shape=[jax.ShapeDtypeStruct((N_PAD, HID), _f32)] * 7,
    )(xp, W0, dis_b)


def _mid_call(part, h, dis_b, selfw_b, b, W):
    return pl.pallas_call(
        _mid_body,
        grid=(GRID,),
        in_specs=[
            pl.BlockSpec((NC, 3, RB, HID), lambda i: (0, 0, i, 0)),
            pl.BlockSpec((RB, HID), lambda i: (i, 0)),
            pl.BlockSpec((3, RB, HID), lambda i: (0, i, 0)),
            pl.BlockSpec((3, RB, HID), lambda i: (0, i, 0)),
            pl.BlockSpec((1, HID), lambda i: (0, 0)),
            pl.BlockSpec((HID, 3 * HID), lambda i: (0, 0)),
        ],
        out_specs=[pl.BlockSpec((RB, HID), lambda i: (i, 0))] * 7,
        out_shape=[jax.ShapeDtypeStruct((N_PAD, HID), _f32)] * 7,
    )(part, h, dis_b, selfw_b, b, W)


def _final_call(part, h, dis_b, selfw_b, b, Wc, bc2):
    return pl.pallas_call(
        _final_body,
        grid=(GRID,),
        in_specs=[
            pl.BlockSpec((NC, 3, RB, HID), lambda i: (0, 0, i, 0)),
            pl.BlockSpec((RB, HID), lambda i: (i, 0)),
            pl.BlockSpec((3, RB, HID), lambda i: (0, i, 0)),
            pl.BlockSpec((3, RB, HID), lambda i: (0, i, 0)),
            pl.BlockSpec((1, HID), lambda i: (0, 0)),
            pl.BlockSpec((OUT, 3 * HID), lambda i: (0, 0)),
            pl.BlockSpec((1, OUT), lambda i: (0, 0)),
        ],
        out_specs=[
            pl.BlockSpec((RB, 3 * HID), lambda i: (i, 0)),
            pl.BlockSpec((RB, OUT), lambda i: (i, 0)),
        ],
        out_shape=[
            jax.ShapeDtypeStruct((N_PAD, 3 * HID), _f32),
            jax.ShapeDtypeStruct((N_PAD, OUT), _f32),
        ],
    )(part, h, dis_b, selfw_b, b, Wc, bc2)


def _prep(edge, w):
    E = edge.shape[1]
    src, dst = edge[0], edge[1]
    mask = src != dst
    ew = jnp.ones((E,), jnp.float32) if w is None else w
    ew = jnp.where(mask, ew, 0.0)
    dst = jnp.where(mask, dst, DUMMY)
    src = jnp.pad(src, (0, E_PAD - E))
    dst = jnp.pad(dst, (0, E_PAD - E), constant_values=DUMMY)
    ew = jnp.pad(ew, (0, E_PAD - E))
    return src, dst, ew


def kernel(x, edge_index, edge_in, edge_out, in_w, out_w,
           W0, W1, W2, b0, b1, b2, Wc, bc):
    sets = [_prep(edge_index, None), _prep(edge_in, in_w),
            _prep(edge_out, out_w)]
    srcA = jnp.stack([t[0] for t in sets]).reshape(3, CK_SET, CW)
    dstA = jnp.stack([t[1] for t in sets]).reshape(3, CK_SET, CW)
    ewA = jnp.stack([t[2] for t in sets]).reshape(3, CK_SET, CW)

    dis_b, selfw_b = _get_deg_kernel()(dstA, ewA)

    xp = jnp.pad(x, ((0, N_PAD - N), (0, 0)))
    h0, g0, g1, g2, g0b, g1b, g2b = _m0_call(xp, W0, dis_b)

    part = _get_prop_kernel()(g0, g1, g2, g0b, g1b, g2b, srcA, dstA, ewA)
    h1, g0, g1, g2, g0b, g1b, g2b = _mid_call(part, h0, dis_b, selfw_b, b0, W1)

    part = _get_prop_kernel()(g0, g1, g2, g0b, g1b, g2b, srcA, dstA, ewA)
    h2, g0, g1, g2, g0b, g1b, g2b = _mid_call(part, h1, dis_b, selfw_b, b1, W2)

    part = _get_prop_kernel()(g0, g1, g2, g0b, g1b, g2b, srcA, dstA, ewA)
    xcat, ls = _final_call(part, h2, dis_b, selfw_b, b2, Wc,
                           bc.reshape(1, OUT))

    return xcat[:N], ls[:N]


# speedup vs baseline: 1.0016x; 1.0016x over previous
import dataclasses
import functools

import jax
import jax.numpy as jnp
from jax import lax
from jax.experimental import pallas as pl
from jax.experimental.pallas import tpu as pltpu
from jax.experimental.pallas import tpu_sc as plsc

N = 10000
D_IN = 128
HID = 128
OUT = 64

N_PAD = 10240
DUMMY = N
NC, NS = 2, 16
CHUNKS = 80
CW = 128
E_PW = CHUNKS * CW
E_PAD = NC * NS * E_PW
ROWS_PS = N_PAD // NS
RB = 512
GRID = N_PAD // RB

CK_SET = E_PAD // CW
C_FAST, C_SLOW = 144, 16
assert NS * (C_FAST + C_SLOW) == CK_SET
assert C_FAST % 2 == 0 and C_SLOW % 2 == 0 and C_SLOW >= 2
FAST_CORE = 0


def _sc_compiler_params():
    cp = pltpu.CompilerParams()
    if "needs_layout_passes" in pltpu.CompilerParams.__dataclass_fields__:
        cp = dataclasses.replace(cp, needs_layout_passes=False)
    return cp


NROW = N_PAD // CW
RPS = NROW // NS


def _rsqrt16(x):
    i = plsc.bitcast(x, jnp.int32)
    y = plsc.bitcast(jnp.int32(0x5F3759DF) - lax.shift_right_logical(i, 1),
                     jnp.float32)
    for _ in range(3):
        y = y * (1.5 - 0.5 * x * y * y)
    return y


@functools.cache
def _get_deg_kernel():
    mesh = plsc.VectorSubcoreMesh(core_axis_name="c", subcore_axis_name="s")
    return functools.partial(
        pl.kernel,
        mesh=mesh,
        compiler_params=_sc_compiler_params(),
        out_type=[jax.ShapeDtypeStruct((3, N_PAD, HID), jnp.float32),
                  jax.ShapeDtypeStruct((3, N_PAD, HID), jnp.float32)],
        scratch_types=[
            pltpu.VMEM((CHUNKS, CW), jnp.int32),
            pltpu.VMEM((CHUNKS, CW), jnp.float32),
            pltpu.VMEM((NROW, CW), jnp.float32),
            pltpu.VMEM((NROW, CW), jnp.float32),
            pltpu.VMEM((NROW, CW), jnp.float32),
            pltpu.VMEM((CW, CW), jnp.float32),
            pltpu.VMEM((3, NROW), jnp.int32),
            pltpu.VMEM_SHARED((3 * NROW, CW), jnp.float32),
        ],
    )(_deg_body)


def _deg_body(dst_hbm, ew_hbm, dis_hbm, selfw_hbm,
              dstv, ewv, a0, a1, a2, bcast, idxv, acc_sh):
    c = lax.axis_index("c")
    s = lax.axis_index("s")
    accs = [a0, a1, a2]

    @pl.loop(0, NROW)
    def _(i):
        z = jnp.zeros((16,), jnp.float32)
        for f in range(8):
            a0[i, pl.ds(16 * f, 16)] = z
            a1[i, pl.ds(16 * f, 16)] = z
            a2[i, pl.ds(16 * f, 16)] = z

    iota = jnp.arange(16, dtype=jnp.int32)
    for k in range(3):
        for m in range(NROW // 16):
            idxv[k, pl.ds(16 * m, 16)] = iota + (k * NROW + 16 * m)

    pltpu.sync_copy(a0.at[pl.ds(0, 15)],
                    acc_sh.at[pl.ds(s * (3 * NROW // NS), 3 * NROW // NS)])
    plsc.subcore_barrier()

    for k in range(3):
        for p in range(2):
            pltpu.sync_copy(
                dst_hbm.at[k, pl.ds(s * 2 * CHUNKS + p * CHUNKS, CHUNKS)],
                dstv)
            pltpu.sync_copy(
                ew_hbm.at[k, pl.ds(s * 2 * CHUNKS + p * CHUNKS, CHUNKS)],
                ewv)

            @pl.loop(0, CHUNKS)
            def _(j):
                for t in range(8):
                    d16 = dstv[j, pl.ds(16 * t, 16)]
                    w16 = ewv[j, pl.ds(16 * t, 16)]
                    plsc.addupdate_scatter(
                        accs[k],
                        [lax.shift_right_logical(d16, 7),
                         jnp.bitwise_and(d16, 127)],
                        w16)

    for k in range(3):
        pltpu.sync_copy(accs[k], acc_sh.at[idxv.at[k]], add=True)
    plsc.subcore_barrier()

    def emit(out_hbm, square):
        for k in range(3):
            pltpu.sync_copy(acc_sh.at[pl.ds(k * NROW + s * RPS, RPS)],
                            ewv.at[pl.ds(0, RPS)])

            @pl.loop(0, RPS)
            def _(r):
                @pl.loop(0, 8)
                def _(f):
                    x = ewv[r, pl.ds(16 * f, 16)] + 1.0
                    y = _rsqrt16(x)
                    if square:
                        y = y * y
                    ewv[CHUNKS - 1, pl.ds(0, 16)] = y

                    @pl.loop(0, 16)
                    def _(l):
                        v = plsc.load_gather(
                            ewv, [jnp.full((16,), CHUNKS - 1, jnp.int32),
                                  jnp.full((16,), l, jnp.int32)])

                        @pl.loop(0, 8)
                        def _(g):
                            bcast[16 * f + l, pl.ds(16 * g, 16)] = v

                pltpu.sync_copy(
                    bcast, out_hbm.at[k, pl.ds((s * RPS + r) * CW, CW)])

    @pl.when(c == FAST_CORE)
    def _():
        emit(dis_hbm, False)

    @pl.when(c != FAST_CORE)
    def _():
        emit(selfw_hbm, True)


@functools.cache
def _get_prop_kernel():
    mesh = plsc.VectorSubcoreMesh(core_axis_name="c", subcore_axis_name="s")
    return functools.partial(
        pl.kernel,
        mesh=mesh,
        compiler_params=_sc_compiler_params(),
        out_type=jax.ShapeDtypeStruct((NC, 3, N_PAD, HID), jnp.float32),
        scratch_types=[
            pltpu.VMEM((2, CW), jnp.int32),
            pltpu.VMEM((2, CW), jnp.int32),
            pltpu.VMEM((2, CW), jnp.float32),
            pltpu.VMEM((CW, HID), jnp.float32),
            pltpu.VMEM((CW, HID), jnp.float32),
            pltpu.VMEM_SHARED((N_PAD, HID), jnp.float32),
            pltpu.SemaphoreType.DMA,
            pltpu.SemaphoreType.DMA,
            pltpu.SemaphoreType.DMA,
            pltpu.SemaphoreType.DMA,
            pltpu.SemaphoreType.DMA,
            pltpu.SemaphoreType.DMA,
        ],
    )(_prop_body)


def _scale_chunk(buf, ewv, b):
    bfull = jnp.full((16,), b, jnp.int32)

    @pl.loop(0, CW)
    def _(e):
        w = plsc.load_gather(ewv, [bfull, jnp.full((16,), e, jnp.int32)])
        for f in range(8):
            sl = (e, pl.ds(16 * f, 16))
            buf[sl] = buf[sl] * w


def _prop_body(hh0, hh1, hh2, hh0b, hh1b, hh2b, src_hbm, dst_hbm, ew_hbm,
               out_hbm, srcv, dstv, ewv, buf0, buf1, acc,
               ss0, ss1, sd0, sd1, sg0, sg1):
    c = lax.axis_index("c")
    s = lax.axis_index("s")
    base = s * ROWS_PS
    bufs = [buf0, buf1]
    sem_src = [ss0, ss1]
    sem_de = [sd0, sd1]
    sem_g = [sg0, sg1]

    for k in range(3):

        def src_copy(j, b, _k=k):
            pltpu.async_copy(src_hbm.at[_k, pl.ds(j, 1)],
                             srcv.at[pl.ds(b, 1)], sem_src[b])

        def src_wait(b):
            pltpu.make_async_copy(src_hbm.at[0, pl.ds(0, 1)],
                                  srcv.at[pl.ds(0, 1)], sem_src[b]).wait()

        def de_copy(j, b, _k=k):
            pltpu.async_copy(dst_hbm.at[_k, pl.ds(j, 1)],
                             dstv.at[pl.ds(b, 1)], sem_de[b])
            if _k > 0:
                pltpu.async_copy(ew_hbm.at[_k, pl.ds(j, 1)],
                                 ewv.at[pl.ds(b, 1)], sem_de[b])

        def de_wait(b, _k=k):
            for _ in range(2 if _k > 0 else 1):
                pltpu.make_async_copy(dst_hbm.at[0, pl.ds(0, 1)],
                                      dstv.at[pl.ds(0, 1)], sem_de[b]).wait()

        def process(b, _k=k):
            if _k > 0:
                _scale_chunk(bufs[b], ewv, b)
            pltpu.sync_copy(bufs[b], acc.at[dstv.at[b]], add=True)

        def pipeline(cbase, nch, hh):
            def gather_start(b):
                pltpu.async_copy(hh.at[srcv.at[b]], bufs[b], sem_g[b])

            def gather_wait(b):
                pltpu.make_async_copy(hh.at[srcv.at[b]], bufs[b],
                                      sem_g[b]).wait()
            src_copy(cbase, 0)
            de_copy(cbase, 0)
            src_copy(cbase + 1, 1)
            de_copy(cbase + 1, 1)
            src_wait(0)
            gather_start(0)

            @pl.loop(0, nch - 2, step=2)
            def _(j):
                gather_wait(0)
                src_wait(1)
                gather_start(1)
                src_copy(cbase + j + 2, 0)
                de_wait(0)
                process(0)
                de_copy(cbase + j + 2, 0)
                gather_wait(1)
                src_wait(0)
                gather_start(0)
                src_copy(cbase + j + 3, 1)
                de_wait(1)
                process(1)
                de_copy(cbase + j + 3, 1)

            gather_wait(0)
            src_wait(1)
            gather_start(1)
            de_wait(0)
            process(0)
            gather_wait(1)
            de_wait(1)
            process(1)

        with jax.named_scope(f"zero{k}"):
            @pl.loop(0, CW)
            def _(r):
                for f in range(8):
                    buf0[r, pl.ds(16 * f, 16)] = jnp.zeros((16,), jnp.float32)

            for m in range(ROWS_PS // CW):
                pltpu.sync_copy(buf0, acc.at[pl.ds(base + m * CW, CW)])
            plsc.subcore_barrier()

        with jax.named_scope(f"pipe{k}"):
            @pl.when(c == FAST_CORE)
            def _():
                pipeline(s * C_FAST, C_FAST, [hh0, hh1, hh2][k])

            @pl.when(c != FAST_CORE)
            def _():
                pipeline(NS * C_FAST + s * C_SLOW, C_SLOW,
                         [hh0b, hh1b, hh2b][k])

            plsc.subcore_barrier()

        with jax.named_scope(f"dump{k}"):
            pltpu.sync_copy(acc.at[pl.ds(base, ROWS_PS)],
                            out_hbm.at[c, k, pl.ds(base, ROWS_PS)])


def _m0_body(x_ref, w0_ref, dis_ref, h_ref, g0_ref, g1_ref, g2_ref,
             g0b_ref, g1b_ref, g2b_ref):
    h = lax.dot_general(x_ref[...], w0_ref[...], (((1,), (1,)), ((), ())),
                        preferred_element_type=jnp.float32)
    dis_b = dis_ref[...]
    h_ref[...] = h
    g0_ref[...] = dis_b[0] * h
    g1_ref[...] = dis_b[1] * h
    g2_ref[...] = dis_b[2] * h
    g0b_ref[...] = dis_b[0] * h
    g1b_ref[...] = dis_b[1] * h
    g2b_ref[...] = dis_b[2] * h


def _combine(part_ref, h_ref, dis_ref, selfw_ref, b_ref):
    p = part_ref[0] + part_ref[1]
    h = h_ref[...]
    xc = dis_ref[...] * p + selfw_ref[...] * h[None] + b_ref[...][None]
    xc = jnp.maximum(xc, 0.0)
    return jnp.concatenate([xc[0], xc[1], xc[2]], axis=-1)


def _mid_body(part_ref, h_ref, dis_ref, selfw_ref, b_ref, w_ref,
              hn_ref, g0_ref, g1_ref, g2_ref, g0b_ref, g1b_ref, g2b_ref):
    xcat = _combine(part_ref, h_ref, dis_ref, selfw_ref, b_ref)
    hn = lax.dot_general(xcat, w_ref[...], (((1,), (1,)), ((), ())),
                         preferred_element_type=jnp.float32)
    hn_ref[...] = hn
    dis_b = dis_ref[...]
    g0_ref[...] = dis_b[0] * hn
    g1_ref[...] = dis_b[1] * hn
    g2_ref[...] = dis_b[2] * hn
    g0b_ref[...] = dis_b[0] * hn
    g1b_ref[...] = dis_b[1] * hn
    g2b_ref[...] = dis_b[2] * hn


def _final_body(part_ref, h_ref, dis_ref, selfw_ref, b_ref, wc_ref, bc_ref,
                xcat_ref, ls_ref):
    xcat = _combine(part_ref, h_ref, dis_ref, selfw_ref, b_ref)
    xcat_ref[...] = xcat
    y = lax.dot_general(xcat, wc_ref[...], (((1,), (1,)), ((), ())),
                        preferred_element_type=jnp.float32)
    y = y + bc_ref[...]
    z = y - jnp.max(y, axis=1, keepdims=True)
    ls_ref[...] = z - jnp.log(jnp.sum(jnp.exp(z), axis=1, keepdims=True))


_f32 = jnp.float32


def _m0_call(xp, W0, dis_b):
    return pl.pallas_call(
        _m0_body,
        grid=(GRID,),
        in_specs=[
            pl.BlockSpec((RB, D_IN), lambda i: (i, 0)),
            pl.BlockSpec((HID, D_IN), lambda i: (0, 0)),
            pl.BlockSpec((3, RB, HID), lambda i: (0, i, 0)),
        ],
        out_specs=[pl.BlockSpec((RB, HID), lambda i: (i, 0))] * 7,
        out_shape=[jax.ShapeDtypeStruct((N_PAD, HID), _f32)] * 7,
    )(xp, W0, dis_b)


def _mid_call(part, h, dis_b, selfw_b, b, W):
    return pl.pallas_call(
        _mid_body,
        grid=(GRID,),
        in_specs=[
            pl.BlockSpec((NC, 3, RB, HID), lambda i: (0, 0, i, 0)),
            pl.BlockSpec((RB, HID), lambda i: (i, 0)),
            pl.BlockSpec((3, RB, HID), lambda i: (0, i, 0)),
            pl.BlockSpec((3, RB, HID), lambda i: (0, i, 0)),
            pl.BlockSpec((1, HID), lambda i: (0, 0)),
            pl.BlockSpec((HID, 3 * HID), lambda i: (0, 0)),
        ],
        out_specs=[pl.BlockSpec((RB, HID), lambda i: (i, 0))] * 7,
        out_shape=[jax.ShapeDtypeStruct((N_PAD, HID), _f32)] * 7,
    )(part, h, dis_b, selfw_b, b, W)


def _final_call(part, h, dis_b, selfw_b, b, Wc, bc2):
    return pl.pallas_call(
        _final_body,
        grid=(GRID,),
        in_specs=[
            pl.BlockSpec((NC, 3, RB, HID), lambda i: (0, 0, i, 0)),
            pl.BlockSpec((RB, HID), lambda i: (i, 0)),
            pl.BlockSpec((3, RB, HID), lambda i: (0, i, 0)),
            pl.BlockSpec((3, RB, HID), lambda i: (0, i, 0)),
            pl.BlockSpec((1, HID), lambda i: (0, 0)),
            pl.BlockSpec((OUT, 3 * HID), lambda i: (0, 0)),
            pl.BlockSpec((1, OUT), lambda i: (0, 0)),
        ],
        out_specs=[
            pl.BlockSpec((RB, 3 * HID), lambda i: (i, 0)),
            pl.BlockSpec((RB, OUT), lambda i: (i, 0)),
        ],
        out_shape=[
            jax.ShapeDtypeStruct((N_PAD, 3 * HID), _f32),
            jax.ShapeDtypeStruct((N_PAD, OUT), _f32),
        ],
    )(part, h, dis_b, selfw_b, b, Wc, bc2)


def _prep(edge, w):
    E = edge.shape[1]
    src, dst = edge[0], edge[1]
    mask = src != dst
    ew = jnp.ones((E,), jnp.float32) if w is None else w
    ew = jnp.where(mask, ew, 0.0)
    dst = jnp.where(mask, dst, DUMMY)
    src = jnp.pad(src, (0, E_PAD - E))
    dst = jnp.pad(dst, (0, E_PAD - E), constant_values=DUMMY)
    ew = jnp.pad(ew, (0, E_PAD - E))
    return src, dst, ew


def kernel(x, edge_index, edge_in, edge_out, in_w, out_w,
           W0, W1, W2, b0, b1, b2, Wc, bc):
    sets = [_prep(edge_index, None), _prep(edge_in, in_w),
            _prep(edge_out, out_w)]
    srcA = jnp.stack([t[0] for t in sets]).reshape(3, CK_SET, CW)
    dstA = jnp.stack([t[1] for t in sets]).reshape(3, CK_SET, CW)
    ewA = jnp.stack([t[2] for t in sets]).reshape(3, CK_SET, CW)

    dis_b, selfw_b = _get_deg_kernel()(dstA, ewA)

    xp = jnp.pad(x, ((0, N_PAD - N), (0, 0)))
    h0, g0, g1, g2, g0b, g1b, g2b = _m0_call(xp, W0, dis_b)

    part = _get_prop_kernel()(g0, g1, g2, g0b, g1b, g2b, srcA, dstA, ewA)
    h1, g0, g1, g2, g0b, g1b, g2b = _mid_call(part, h0, dis_b, selfw_b, b0, W1)

    part = _get_prop_kernel()(g0, g1, g2, g0b, g1b, g2b, srcA, dstA, ewA)
    h2, g0, g1, g2, g0b, g1b, g2b = _mid_call(part, h1, dis_b, selfw_b, b1, W2)

    part = _get_prop_kernel()(g0, g1, g2, g0b, g1b, g2b, srcA, dstA, ewA)
    xcat, ls = _final_call(part, h2, dis_b, selfw_b, b2, Wc,
                           bc.reshape(1, OUT))

    return xcat[:N], ls[:N]
